# baseline (speedup 1.0000x reference)
import functools

import jax
import jax.numpy as jnp
from jax import lax
from jax.experimental import pallas as pl
from jax.experimental.pallas import tpu as pltpu
from jax.experimental.pallas import tpu_sc as plsc

T, H, I, E, K = 2048, 2048, 1536, 8, 2
NE = E + 1
EPAD = 128
TB = 256
TI = 512
NI = I // TI
NT = T // TB
NA = T * K
NP = T + NA + E * TB
NBLK = NP // TB
NC, NS = 2, 16
NW = NC * NS


def _router_body(x_ref, gw_ref, r_ref):
    xb = x_ref[...]
    gw = gw_ref[...]
    logits = lax.dot_general(xb, gw, (((1,), (1,)), ((), ())),
                             preferred_element_type=jnp.float32)
    lane = lax.broadcasted_iota(jnp.int32, (TB, EPAD), 1)
    neg = jnp.float32(-1e30)
    l = jnp.where(lane < E, logits, neg)
    m0 = jnp.max(l, axis=1, keepdims=True)
    i0 = jnp.min(jnp.where(l == m0, lane, EPAD), axis=1, keepdims=True)
    l2 = jnp.where(lane == i0, neg, l)
    m1 = jnp.max(l2, axis=1, keepdims=True)
    i1 = jnp.min(jnp.where(l2 == m1, lane, EPAD), axis=1, keepdims=True)
    w0 = jax.nn.sigmoid(m0 - m1)
    r_ref[...] = (jnp.where(lane == 0, i0.astype(jnp.float32), 0.0)
                  + jnp.where(lane == 1, i1.astype(jnp.float32), 0.0)
                  + jnp.where(lane == 2, w0, 0.0)
                  + jnp.where(lane == 3, 1.0 - w0, 0.0))


def _reg_gather(vec, idx):
    return jnp.take_along_axis(vec, idx, axis=0, mode="promise_in_bounds")


def _meta_body(eflat_hbm, pos_hbm, rt_hbm, gid_hbm, ids_v, pos_v, rt_v, gid_v):
    wid = lax.axis_index("s") * NC + lax.axis_index("c")
    lanei = lax.iota(jnp.int32, 16)

    @pl.when(wid == 0)
    def _():
        zero16 = jnp.zeros((16,), jnp.int32)

        def zb(j, c):
            rt_v[pl.ds(j * 16, 16)] = zero16
            return c
        lax.fori_loop(0, NP // 16, zb, 0)

        def ib(j, c):
            rt_v[pl.ds(j * 16, 16)] = j * 16 + lanei
            return c
        lax.fori_loop(0, T // 16, ib, 0)

        pltpu.sync_copy(eflat_hbm, ids_v)

        def cb(j, cntv):
            idv = ids_v[pl.ds(j * 16, 16)]
            for e in range(E):
                c = plsc.all_reduce_population_count(idv == e)
                cntv = cntv + jnp.where(lanei == e, c, 0)
            return cntv
        cntv = lax.fori_loop(0, NA // 16, cb, zero16)

        blkv = ((cntv + TB - 1) >> 8) << 8
        startsv = T + plsc.cumsum(blkv) - blkv
        endsv = startsv + blkv

        for half in range(2):
            rowstart = (lanei + 16 * half) * TB
            g = zero16
            for e in range(E):
                efull = jnp.full((16,), e, jnp.int32)
                st = _reg_gather(startsv, efull)
                en = _reg_gather(endsv, efull)
                m = jnp.logical_and(rowstart >= st, rowstart < en)
                g = jnp.where(m, e + 1, g)
            gid_v[pl.ds(16 * half, 16)] = g

        def rb(j, runv):
            idv = ids_v[pl.ds(j * 16, 16)]
            st_g = _reg_gather(startsv, idv)
            run_g = _reg_gather(runv, idv)
            rankv = zero16
            addv = zero16
            for e in range(E):
                m = idv == e
                r = plsc.cumsum(m.astype(jnp.int32))
                rankv = jnp.where(m, r - 1, rankv)
                addv = addv + jnp.where(
                    lanei == e, plsc.all_reduce_population_count(m), 0)
            posv = st_g + run_g + rankv
            pos_v[pl.ds(j * 16, 16)] = posv
            tok = (j * 16 + lanei) >> 1
            plsc.store_scatter(rt_v, [posv], tok)
            return runv + addv
        lax.fori_loop(0, NA // 16, rb, zero16)

        pltpu.sync_copy(pos_v, pos_hbm)
        pltpu.sync_copy(rt_v, rt_hbm)
        pltpu.sync_copy(gid_v, gid_hbm)


def _gather_body(x_hbm, rt_hbm, xs_hbm, idx_v, rows_v, sem):
    wid = lax.axis_index("s") * NC + lax.axis_index("c")
    rows_per_w = NP // NW
    for k in range(4):
        base = wid * rows_per_w + k * 64
        pltpu.sync_copy(rt_hbm.at[pl.ds(base, 64)], idx_v)
        pltpu.async_copy(x_hbm.at[idx_v], rows_v, sem).wait()
        pltpu.sync_copy(rows_v, xs_hbm.at[pl.ds(base, 64)])


def _gemm_body(gid_ref, x_ref, wg_ref, wu_ref, wd_ref, out_ref):
    i = pl.program_id(1)
    xb = x_ref[...]
    g = lax.dot_general(xb, wg_ref[0], (((1,), (1,)), ((), ())),
                        preferred_element_type=jnp.float32)
    u = lax.dot_general(xb, wu_ref[0], (((1,), (1,)), ((), ())),
                        preferred_element_type=jnp.float32)
    h = (g * jax.nn.sigmoid(g) * u).astype(jnp.bfloat16)
    partial = lax.dot_general(h, wd_ref[0], (((1,), (1,)), ((), ())),
                              preferred_element_type=jnp.float32)

    @pl.when(i == 0)
    def _init():
        out_ref[...] = partial

    @pl.when(i != 0)
    def _acc():
        out_ref[...] += partial


def _combine_body(y_hbm, pos_hbm, wf_hbm, out_hbm, pidx_v, rows_v, sh_v,
                  w_v, out_v, sem):
    wid = lax.axis_index("s") * NC + lax.axis_index("c")
    lanei = lax.iota(jnp.int32, 16)
    toks_per_w = T // NW

    def chunk(c, carry):
        tbase = wid * toks_per_w + c * 8
        pltpu.sync_copy(pos_hbm.at[pl.ds(2 * tbase, 16)], pidx_v)
        pltpu.async_copy(y_hbm.at[pidx_v], rows_v, sem).wait()
        pltpu.sync_copy(y_hbm.at[pl.ds(tbase, 8)], sh_v)
        pltpu.sync_copy(wf_hbm.at[pl.ds(2 * tbase, 16)], w_v)
        wv = w_v[...]
        for i in range(8):
            w0 = _reg_gather(wv, jnp.full((16,), 2 * i, jnp.int32))
            w1 = _reg_gather(wv, jnp.full((16,), 2 * i + 1, jnp.int32))

            def vb(v, cc):
                sl = pl.ds(v * 16, 16)
                out_v[i, sl] = (w0 * rows_v[2 * i, sl]
                                + w1 * rows_v[2 * i + 1, sl]
                                + sh_v[i, sl])
                return cc
            lax.fori_loop(0, H // 16, vb, 0)
        pltpu.sync_copy(out_v, out_hbm.at[pl.ds(tbase, 8)])
        return carry
    lax.fori_loop(0, toks_per_w // 8, chunk, 0)


_SC_MESH = plsc.VectorSubcoreMesh(core_axis_name="c", subcore_axis_name="s",
                                  num_cores=NC, num_subcores=NS)


@jax.jit
def kernel(x, gate_w, Wg, Wu, Wd, sg, su, sd):
    gw_pad = jnp.zeros((EPAD, H), jnp.float32).at[:E].set(gate_w)
    routed = pl.pallas_call(
        _router_body,
        grid=(NT,),
        in_specs=[
            pl.BlockSpec((TB, H), lambda t: (t, 0)),
            pl.BlockSpec((EPAD, H), lambda t: (0, 0)),
        ],
        out_specs=pl.BlockSpec((TB, EPAD), lambda t: (t, 0)),
        out_shape=jax.ShapeDtypeStruct((T, EPAD), jnp.float32),
    )(x, gw_pad)

    eflat = routed[:, :K].astype(jnp.int32).reshape(NA)
    wflat = routed[:, K:2 * K].reshape(NA)

    pos, row_token, gid = pl.kernel(
        _meta_body,
        out_type=(
            jax.ShapeDtypeStruct((NA,), jnp.int32),
            jax.ShapeDtypeStruct((NP,), jnp.int32),
            jax.ShapeDtypeStruct((NBLK,), jnp.int32),
        ),
        mesh=_SC_MESH,
        compiler_params=pltpu.CompilerParams(needs_layout_passes=False),
        scratch_types=[
            pltpu.VMEM((NA,), jnp.int32),
            pltpu.VMEM((NA,), jnp.int32),
            pltpu.VMEM((NP,), jnp.int32),
            pltpu.VMEM((NBLK,), jnp.int32),
        ],
    )(eflat)

    x16 = x.astype(jnp.bfloat16)
    x32 = lax.bitcast_convert_type(x16.reshape(T, H // 2, 2), jnp.int32)
    xs32 = pl.kernel(
        _gather_body,
        out_type=jax.ShapeDtypeStruct((NP, H // 2), jnp.int32),
        mesh=_SC_MESH,
        compiler_params=pltpu.CompilerParams(needs_layout_passes=False),
        scratch_types=[
            pltpu.VMEM((64,), jnp.int32),
            pltpu.VMEM((64, H // 2), jnp.int32),
            pltpu.SemaphoreType.DMA,
        ],
    )(x32, row_token)
    xs16 = lax.bitcast_convert_type(xs32, jnp.bfloat16).reshape(NP, H)

    wg_all = jnp.concatenate([sg[None], Wg], axis=0).astype(jnp.bfloat16)
    wu_all = jnp.concatenate([su[None], Wu], axis=0).astype(jnp.bfloat16)
    wd_all = jnp.concatenate([sd[None], Wd], axis=0).astype(jnp.bfloat16)

    y = pl.pallas_call(
        _gemm_body,
        grid_spec=pltpu.PrefetchScalarGridSpec(
            num_scalar_prefetch=1,
            grid=(NBLK, NI),
            in_specs=[
                pl.BlockSpec((TB, H), lambda b, i, gid_ref: (b, 0)),
                pl.BlockSpec((1, TI, H), lambda b, i, gid_ref: (gid_ref[b], i, 0)),
                pl.BlockSpec((1, TI, H), lambda b, i, gid_ref: (gid_ref[b], i, 0)),
                pl.BlockSpec((1, H, TI), lambda b, i, gid_ref: (gid_ref[b], 0, i)),
            ],
            out_specs=pl.BlockSpec((TB, H), lambda b, i, gid_ref: (b, 0)),
        ),
        out_shape=jax.ShapeDtypeStruct((NP, H), jnp.float32),
    )(gid, xs16, wg_all, wu_all, wd_all)

    out = pl.kernel(
        _combine_body,
        out_type=jax.ShapeDtypeStruct((T, H), jnp.float32),
        mesh=_SC_MESH,
        compiler_params=pltpu.CompilerParams(needs_layout_passes=False),
        scratch_types=[
            pltpu.VMEM((16,), jnp.int32),
            pltpu.VMEM((16, H), jnp.float32),
            pltpu.VMEM((8, H), jnp.float32),
            pltpu.VMEM((16,), jnp.float32),
            pltpu.VMEM((8, H), jnp.float32),
            pltpu.SemaphoreType.DMA,
        ],
    )(y, pos, wflat)
    return out
